# baseline (speedup 1.0000x reference)
import functools

import jax
import jax.numpy as jnp
from jax.experimental import pallas as pl
from jax.experimental.pallas import tpu as pltpu

B, N, C_IN, H, C_OUT = 16384, 64, 4, 32, 16
NH = N // 2
BLOCK_B = 2048


def _fused_mlp_pool_kernel(x_ref, w1_ref, b1_ref, w2_ref, b2_ref, out_ref,
                           a_scr, nb1_scr, m_scr, bias_scr):
    @pl.when(pl.program_id(0) == 0)
    def _prep():
        w1t = jnp.tile(w1_ref[...], (NH, H))
        rows = jax.lax.broadcasted_iota(jnp.int32, (NH * C_IN, NH * H), 0)
        cols = jax.lax.broadcasted_iota(jnp.int32, (NH * C_IN, NH * H), 1)
        a_scr[...] = jnp.where(rows // C_IN == cols // H, w1t, 0.0)
        nb1_scr[...] = jnp.tile(-b1_ref[...], (1, NH))
        m_scr[...] = jnp.tile(w2_ref[...], (NH, 1)) * (1.0 / N)
        bias_scr[...] = (
            jnp.dot(b1_ref[...], w2_ref[...], preferred_element_type=jnp.float32)
            + b2_ref[...]
        )

    xw = x_ref[...]
    p = None
    for half in range(2):
        xh = xw[:, half * 128:(half + 1) * 128].astype(jnp.float32)
        h = jnp.dot(xh, a_scr[...], preferred_element_type=jnp.float32)
        h = jnp.maximum(h, nb1_scr[...])
        ph = jnp.dot(h, m_scr[...], preferred_element_type=jnp.float32)
        p = ph if p is None else p + ph
    out_ref[...] = p + bias_scr[...]


@functools.partial(jax.jit, static_argnames=())
def kernel(x, W1, b1, W2, b2):
    x2d = x.astype(jnp.bfloat16).reshape(B, N * C_IN)

    grid = (B // BLOCK_B,)
    return pl.pallas_call(
        _fused_mlp_pool_kernel,
        grid=grid,
        in_specs=[
            pl.BlockSpec((BLOCK_B, N * C_IN), lambda i: (i, 0)),
            pl.BlockSpec((C_IN, H), lambda i: (0, 0)),
            pl.BlockSpec((1, H), lambda i: (0, 0)),
            pl.BlockSpec((H, C_OUT), lambda i: (0, 0)),
            pl.BlockSpec((1, C_OUT), lambda i: (0, 0)),
        ],
        scratch_shapes=[
            pltpu.VMEM((NH * C_IN, NH * H), jnp.float32),
            pltpu.VMEM((1, NH * H), jnp.float32),
            pltpu.VMEM((NH * H, C_OUT), jnp.float32),
            pltpu.VMEM((1, C_OUT), jnp.float32),
        ],
        out_specs=pl.BlockSpec((BLOCK_B, C_OUT), lambda i: (i, 0)),
        out_shape=jax.ShapeDtypeStruct((B, C_OUT), jnp.float32),
        compiler_params=pltpu.CompilerParams(
            dimension_semantics=("arbitrary",),
        ),
    )(x2d, W1, b1.reshape(1, H), W2, b2.reshape(1, C_OUT))


# speedup vs baseline: 1.0123x; 1.0123x over previous
import functools

import jax
import jax.numpy as jnp
from jax.experimental import pallas as pl
from jax.experimental.pallas import tpu as pltpu

B, N, C_IN, H, C_OUT = 16384, 64, 4, 32, 16
NH = N // 2
BLOCK_B = 2048


def _fused_mlp_pool_kernel(x_ref, w1_ref, b1_ref, w2_ref, b2_ref, out_ref,
                           a_scr, nb1_scr, m_scr, bias_scr):
    @pl.when(pl.program_id(0) == 0)
    def _prep():
        w1t = jnp.tile(w1_ref[...], (NH, H))
        rows = jax.lax.broadcasted_iota(jnp.int32, (NH * C_IN, NH * H), 0)
        cols = jax.lax.broadcasted_iota(jnp.int32, (NH * C_IN, NH * H), 1)
        a_scr[...] = jnp.where(rows // C_IN == cols // H, w1t, 0.0)
        nb1_scr[...] = jnp.tile(-b1_ref[...], (1, NH))
        m_scr[...] = jnp.tile(w2_ref[...], (NH, 1)) * (1.0 / N)
        bias_scr[...] = (
            jnp.dot(b1_ref[...], w2_ref[...], preferred_element_type=jnp.float32)
            + b2_ref[...]
        )

    xw = x_ref[...]
    p = None
    for half in range(2):
        xh = xw[:, half * 128:(half + 1) * 128]
        h = jnp.dot(xh, a_scr[...], preferred_element_type=jnp.float32)
        h = jnp.maximum(h, nb1_scr[...])
        ph = jnp.dot(h, m_scr[...], preferred_element_type=jnp.float32)
        p = ph if p is None else p + ph
    out_ref[...] = p + bias_scr[...]


@functools.partial(jax.jit, static_argnames=())
def kernel(x, W1, b1, W2, b2):
    x2d = x.reshape(B, N * C_IN)

    grid = (B // BLOCK_B,)
    return pl.pallas_call(
        _fused_mlp_pool_kernel,
        grid=grid,
        in_specs=[
            pl.BlockSpec((BLOCK_B, N * C_IN), lambda i: (i, 0)),
            pl.BlockSpec((C_IN, H), lambda i: (0, 0)),
            pl.BlockSpec((1, H), lambda i: (0, 0)),
            pl.BlockSpec((H, C_OUT), lambda i: (0, 0)),
            pl.BlockSpec((1, C_OUT), lambda i: (0, 0)),
        ],
        scratch_shapes=[
            pltpu.VMEM((NH * C_IN, NH * H), jnp.float32),
            pltpu.VMEM((1, NH * H), jnp.float32),
            pltpu.VMEM((NH * H, C_OUT), jnp.float32),
            pltpu.VMEM((1, C_OUT), jnp.float32),
        ],
        out_specs=pl.BlockSpec((BLOCK_B, C_OUT), lambda i: (i, 0)),
        out_shape=jax.ShapeDtypeStruct((B, C_OUT), jnp.float32),
        compiler_params=pltpu.CompilerParams(
            dimension_semantics=("arbitrary",),
        ),
    )(x2d, W1, b1.reshape(1, H), W2, b2.reshape(1, C_OUT))
